# Initial kernel scaffold; baseline (speedup 1.0000x reference)
import jax
import jax.numpy as jnp
from jax.experimental import pallas as pl


def kernel(node_feat, edge_feat, x_indices1, x_indices2, mask_valid, W_e2n, b_e2n, w_gate, w_self, b_edge):
    raise NotImplementedError("write your pallas kernel here")


# speedup vs baseline: 3.2425x; 3.2425x over previous
import functools

import jax
import jax.numpy as jnp
from jax import lax
from jax.experimental import pallas as pl
from jax.experimental.pallas import tpu as pltpu
from jax.experimental.pallas import tpu_sc as plsc

N = 1024
E = 523776
D = 64
NW = 32
SC = 512
NSC = E // SC
ACC_ROWS = N + 8
ROWS_PER_SUB = N // 16
TRASH = N

_mesh = plsc.VectorSubcoreMesh(core_axis_name="c", subcore_axis_name="s")


def _wid():
    return lax.axis_index("s") * 2 + lax.axis_index("c")


def _span(wid):
    base = wid * (32 * SC)
    nsc = jnp.where(wid < 31, 32, 31)
    return base, nsc


@functools.partial(
    pl.kernel,
    out_type=jax.ShapeDtypeStruct((2, N, D), jnp.float32),
    mesh=_mesh,
    scratch_types=[
        pltpu.VMEM((SC, D), jnp.float32),
        pltpu.VMEM((SC,), jnp.int32),
        pltpu.VMEM((SC,), jnp.int32),
        pltpu.VMEM((SC,), jnp.float32),
    ] + [pltpu.VMEM((128,), jnp.int32) for _ in range(8)] + [
        pltpu.VMEM((ROWS_PER_SUB, D), jnp.float32),
        pltpu.VMEM_SHARED((ACC_ROWS, D), jnp.float32),
    ],
    compiler_params=pltpu.CompilerParams(use_tc_tiling_on_sc=False),
)
def _scatter_kernel(edge_hbm, idx1_hbm, idx2_hbm, mask_hbm, out_hbm,
                    rows_v, i1_v, i2_v, m_v,
                    ea0, ea1, ea2, ea3, eb0, eb1, eb2, eb3, z_v, acc_sh):
    e1_refs = [ea0, ea1, ea2, ea3]
    e2_refs = [eb0, eb1, eb2, eb3]
    c = lax.axis_index("c")
    s = lax.axis_index("s")
    wid = _wid()
    base, nsc = _span(wid)

    zvec = jnp.zeros((16,), jnp.float32)
    for r in range(ROWS_PER_SUB):
        for q in range(4):
            z_v[r, pl.ds(q * 16, 16)] = zvec
    pltpu.sync_copy(z_v, acc_sh.at[pl.ds(s * ROWS_PER_SUB, ROWS_PER_SUB)])
    plsc.subcore_barrier()

    def body(j, _):
        off = base + j * SC
        pltpu.sync_copy(edge_hbm.at[pl.ds(off, SC)], rows_v)
        pltpu.sync_copy(idx1_hbm.at[pl.ds(off, SC)], i1_v)
        pltpu.sync_copy(idx2_hbm.at[pl.ds(off, SC)], i2_v)
        pltpu.sync_copy(mask_hbm.at[pl.ds(off, SC)], m_v)
        for j2 in range(4):
            for k in range(8):
                g = j2 * 128 + k * 16
                keep = m_v[pl.ds(g, 16)] > 0.0
                e1_refs[j2][pl.ds(k * 16, 16)] = jnp.where(
                    keep, i1_v[pl.ds(g, 16)], TRASH)
                e2_refs[j2][pl.ds(k * 16, 16)] = jnp.where(
                    keep, i2_v[pl.ds(g, 16)], TRASH)
        for j2 in range(4):
            blk = rows_v.at[pl.ds(j2 * 128, 128)]
            pltpu.sync_copy(blk, acc_sh.at[e1_refs[j2]], add=True)
            pltpu.sync_copy(blk, acc_sh.at[e2_refs[j2]], add=True)
        return ()

    lax.fori_loop(0, nsc, body, ())
    plsc.subcore_barrier()
    pltpu.sync_copy(acc_sh.at[pl.ds(s * ROWS_PER_SUB, ROWS_PER_SUB)],
                    out_hbm.at[c].at[pl.ds(s * ROWS_PER_SUB, ROWS_PER_SUB)])


def _node_body(agg_ref, nf_ref, w_ref, b_ref, out_ref):
    h = jnp.tanh(
        jax.lax.dot(agg_ref[...], w_ref[...],
                    preferred_element_type=jnp.float32)
        + b_ref[0:1, :])
    out_ref[...] = nf_ref[...] + h


_node_call = pl.pallas_call(
    _node_body,
    out_shape=jax.ShapeDtypeStruct((N, D), jnp.float32),
)


@functools.partial(
    pl.kernel,
    out_type=jax.ShapeDtypeStruct((E, D), jnp.float32),
    mesh=_mesh,
    scratch_types=[
        pltpu.VMEM((SC, D), jnp.float32),
        pltpu.VMEM((SC, D), jnp.float32),
        pltpu.VMEM((SC, D), jnp.float32),
        pltpu.VMEM((SC,), jnp.int32),
        pltpu.VMEM((SC,), jnp.int32),
        pltpu.VMEM((SC,), jnp.float32),
        pltpu.VMEM((D,), jnp.float32),
        pltpu.VMEM((D,), jnp.float32),
        pltpu.VMEM((D,), jnp.float32),
        pltpu.SemaphoreType.DMA,
    ],
    compiler_params=pltpu.CompilerParams(use_tc_tiling_on_sc=False),
)
def _edge_kernel(node_hbm, edge_hbm, idx1_hbm, idx2_hbm, mask_hbm,
                 wg_hbm, ws_hbm, be_hbm, out_hbm,
                 ef_v, n1_v, n2_v, i1_v, i2_v, m_v, wg_v, ws_v, be_v, sem):
    wid = _wid()
    base, nsc = _span(wid)

    pltpu.sync_copy(wg_hbm, wg_v)
    pltpu.sync_copy(ws_hbm, ws_v)
    pltpu.sync_copy(be_hbm, be_v)
    wg = [wg_v[pl.ds(q * 16, 16)] for q in range(4)]
    ws = [ws_v[pl.ds(q * 16, 16)] for q in range(4)]
    be = [be_v[pl.ds(q * 16, 16)] for q in range(4)]

    def body(j, _):
        off = base + j * SC
        pltpu.sync_copy(idx1_hbm.at[pl.ds(off, SC)], i1_v)
        pltpu.sync_copy(idx2_hbm.at[pl.ds(off, SC)], i2_v)
        pltpu.sync_copy(mask_hbm.at[pl.ds(off, SC)], m_v)
        pltpu.sync_copy(edge_hbm.at[pl.ds(off, SC)], ef_v)
        handles = []
        for t in range(4):
            sl = pl.ds(t * 128, 128)
            handles.append(pltpu.async_copy(
                node_hbm.at[i1_v.at[sl]], n1_v.at[sl], sem))
            handles.append(pltpu.async_copy(
                node_hbm.at[i2_v.at[sl]], n2_v.at[sl], sem))
        for h in handles:
            h.wait()

        def gbody(g, _):
            m16 = m_v[pl.ds(g * 16, 16)]
            for i in range(16):
                e = g * 16 + i
                m = m16[i]
                for q in range(4):
                    sl = pl.ds(q * 16, 16)
                    a = n1_v[e, sl] + n2_v[e, sl]
                    z = a * wg[q] + ef_v[e, sl] * ws[q] + be[q]
                    t = 1.0 - 2.0 / (jnp.exp(2.0 * z) + 1.0)
                    ef_v[e, sl] = t * m
            return ()

        lax.fori_loop(0, SC // 16, gbody, ())
        pltpu.sync_copy(ef_v, out_hbm.at[pl.ds(off, SC)])
        return ()

    lax.fori_loop(0, nsc, body, ())


def kernel(node_feat, edge_feat, x_indices1, x_indices2, mask_valid,
           W_e2n, b_e2n, w_gate, w_self, b_edge):
    edge2d = edge_feat.reshape(E, D)
    mask1d = mask_valid.reshape(E)
    node2d = node_feat.reshape(N, D)

    partials = _scatter_kernel(edge2d, x_indices1, x_indices2, mask1d)
    agg = partials[0] + partials[1]
    node_out2d = _node_call(agg, node2d, W_e2n,
                            jnp.tile(b_e2n.reshape(1, D), (8, 1)))
    edge_out2d = _edge_kernel(node_out2d, edge2d, x_indices1, x_indices2,
                              mask1d, w_gate, w_self, b_edge)
    return (node_out2d.reshape(1, N, D), edge_out2d.reshape(1, E, D))
